# baseline (speedup 1.0000x reference)
import jax
import jax.numpy as jnp
from jax import lax
from jax.experimental import pallas as pl
from jax.experimental.pallas import tpu as pltpu
from jax.experimental.pallas import tpu_sc as plsc

N = 10000
D = 128
E = 320000
NC, NS = 2, 16
NW = NC * NS
C = 88
CH0 = 64
CH1 = 164
CHMAX = max(CH0, CH1)
E_PAD = NS * (CH0 + CH1) * C
N_PAD = 10112
NPT = N_PAD // NS
WCH = [C] * 7 + [16]


def _sc_body(aux_hbm, x_hbm, parts_hbm,
             acc, g0, g1, s0, s1, a0, a1, a2, a3,
             semg0, semg1, sems0, sems1, semi0, semi1, semi2, semi3):
    c = lax.axis_index("c")
    s = lax.axis_index("s")
    wid = c * NS + s
    ncv = jnp.where(c == 0, CH0, CH1)
    gbuf = (g0, g1)
    sbuf = (s0, s1)
    aux = (a0, a1, a2, a3)
    semg = (semg0, semg1)
    sems = (sems0, sems1)
    semi = (semi0, semi1, semi2, semi3)

    zero16 = jnp.zeros((16,), jnp.float32)

    def vbcast(vec, e):
        idx = jnp.full((16, 1), e, jnp.int32)
        dn = lax.GatherDimensionNumbers(offset_dims=(),
                                        collapsed_slice_dims=(0,),
                                        start_index_map=(0,))
        return lax.gather(vec, idx, dn, (1,),
                          mode=lax.GatherScatterMode.PROMISE_IN_BOUNDS)

    def zero_row(r, carry):
        for j in range(D // 16):
            g0[r, pl.ds(j * 16, 16)] = zero16
        return carry

    lax.fori_loop(0, C, zero_row, 0)

    off = 0
    for sz in WCH:
        pltpu.sync_copy(g0.at[pl.ds(0, sz)],
                        acc.at[pl.ds(s * NPT + off, sz)])
        off += sz
    plsc.subcore_barrier()

    def idx_fetch(i, q):
        pltpu.async_copy(aux_hbm.at[wid, i], aux[q], semi[q])

    def idx_wait(i, q):
        pltpu.make_async_copy(aux_hbm.at[wid, i], aux[q], semi[q]).wait()

    def gather_issue(b, q):
        pltpu.async_copy(x_hbm.at[aux[q].at[1]], gbuf[b], semg[b])

    def body(i, phase):
        b, q = phase % 2, phase
        b1, q1 = (phase + 1) % 2, (phase + 1) % 4
        pltpu.make_async_copy(x_hbm.at[aux[q].at[1]], gbuf[b], semg[b]).wait()

        @pl.when(i >= 2)
        def _():
            pltpu.make_async_copy(sbuf[b], acc.at[aux[q].at[0]], sems[b]).wait()

        @pl.when(i + 1 <= ncv - 1)
        def _():
            idx_wait(i + 1, q1)
            gather_issue(b1, q1)

        @pl.when((i >= 2) & (i + 2 <= ncv - 1))
        def _():
            idx_fetch(i + 2, (phase + 2) % 4)

        def scale_group(gidx, carry2):
            base = jnp.minimum(gidx * 16, C - 16)
            w16 = lax.bitcast_convert_type(aux[q][2, pl.ds(base, 16)],
                                           jnp.float32)
            for e in range(16):
                wb = vbcast(w16, e)
                r = base + e
                for j in range(D // 16):
                    sl = pl.ds(j * 16, 16)
                    sbuf[b][r, sl] = gbuf[b][r, sl] * wb
            return carry2

        lax.fori_loop(0, (C + 15) // 16, scale_group, 0)
        pltpu.async_copy(sbuf[b], acc.at[aux[q].at[0]], sems[b], add=True)

    for i in range(4):
        idx_fetch(i, i)
    idx_wait(0, 0)
    gather_issue(0, 0)

    def outer(kk, carry):
        for u in range(4):
            body(kk * 4 + u, u)
        return carry

    lax.fori_loop(0, ncv // 4, outer, 0)

    pltpu.make_async_copy(sbuf[0], acc.at[aux[0].at[0]], sems[0]).wait()
    pltpu.make_async_copy(sbuf[1], acc.at[aux[1].at[0]], sems[1]).wait()

    plsc.subcore_barrier()

    off = 0
    for sz in WCH:
        rb = s * NPT + off
        pltpu.sync_copy(acc.at[pl.ds(rb, sz)], g0.at[pl.ds(0, sz)])
        pltpu.sync_copy(g0.at[pl.ds(0, sz)], parts_hbm.at[c, pl.ds(rb, sz)])
        off += sz


def _sc_scatter(aux, x):
    mesh = plsc.VectorSubcoreMesh(core_axis_name="c", subcore_axis_name="s",
                                  num_cores=NC, num_subcores=NS)
    return pl.kernel(
        _sc_body,
        out_type=jax.ShapeDtypeStruct((NC, N_PAD, D), jnp.float32),
        mesh=mesh,
        scratch_types=[
            pltpu.VMEM_SHARED((N_PAD, D), jnp.float32),
            pltpu.VMEM((C, D), jnp.float32),
            pltpu.VMEM((C, D), jnp.float32),
            pltpu.VMEM((C, D), jnp.float32),
            pltpu.VMEM((C, D), jnp.float32),
            pltpu.VMEM((3, C), jnp.int32),
            pltpu.VMEM((3, C), jnp.int32),
            pltpu.VMEM((3, C), jnp.int32),
            pltpu.VMEM((3, C), jnp.int32),
            pltpu.SemaphoreType.DMA,
            pltpu.SemaphoreType.DMA,
            pltpu.SemaphoreType.DMA,
            pltpu.SemaphoreType.DMA,
            pltpu.SemaphoreType.DMA,
            pltpu.SemaphoreType.DMA,
            pltpu.SemaphoreType.DMA,
            pltpu.SemaphoreType.DMA,
        ],
    )(aux, x)


def _combine_body(p_ref, d_ref, o_ref):
    o_ref[...] = (p_ref[0] + p_ref[1]) * d_ref[...]


def _combine(parts, deg2d):
    bn = 2000
    return pl.pallas_call(
        _combine_body,
        out_shape=jax.ShapeDtypeStruct((N, D), jnp.float32),
        grid=(N // bn,),
        in_specs=[
            pl.BlockSpec((NC, bn, D), lambda i: (0, i, 0)),
            pl.BlockSpec((bn, 1), lambda i: (i, 0)),
        ],
        out_specs=pl.BlockSpec((bn, D), lambda i: (i, 0)),
    )(parts, deg2d)


def kernel(x, edge_index, edge_weight, deg_inv):
    row = edge_index[0].astype(jnp.int32)
    col = edge_index[1].astype(jnp.int32)
    wbits = lax.bitcast_convert_type(edge_weight.astype(jnp.float32), jnp.int32)
    pad = E_PAD - E
    zpad = jnp.zeros((pad,), jnp.int32)
    n0 = NS * CH0 * C

    def split(arr):
        arrp = jnp.concatenate([arr, zpad])
        return (arrp[:n0].reshape(NS, CH0, C), arrp[n0:].reshape(NS, CH1, C))

    r0, r1 = split(row)
    c0, c1 = split(col)
    w0, w1 = split(wbits)
    aux0 = jnp.stack([r0, c0, w0], axis=2)
    aux1 = jnp.stack([r1, c1, w1], axis=2)
    aux = jnp.zeros((NW, CHMAX, 3, C), jnp.int32)
    aux = aux.at[:NS, :CH0].set(aux0).at[NS:, :CH1].set(aux1)
    parts = _sc_scatter(aux, x)
    return _combine(parts, deg_inv[:, None])


# speedup vs baseline: 1.0326x; 1.0326x over previous
import jax
import jax.numpy as jnp
from jax import lax
from jax.experimental import pallas as pl
from jax.experimental.pallas import tpu as pltpu
from jax.experimental.pallas import tpu_sc as plsc

N = 10000
D = 128
E = 320000
NC, NS = 2, 16
NW = NC * NS
C = 88
CH0 = 72
CH1 = 156
CHMAX = max(CH0, CH1)
E_PAD = NS * (CH0 + CH1) * C
N_PAD = 10112
NPT = N_PAD // NS
WCH = [C] * 7 + [16]


def _sc_body(aux_hbm, x_hbm, parts_hbm,
             acc, g0, g1, s0, s1, a0, a1, a2, a3,
             semg0, semg1, sems0, sems1, semi0, semi1, semi2, semi3):
    c = lax.axis_index("c")
    s = lax.axis_index("s")
    wid = c * NS + s
    ncv = jnp.where(c == 0, CH0, CH1)
    gbuf = (g0, g1)
    sbuf = (s0, s1)
    aux = (a0, a1, a2, a3)
    semg = (semg0, semg1)
    sems = (sems0, sems1)
    semi = (semi0, semi1, semi2, semi3)

    zero16 = jnp.zeros((16,), jnp.float32)

    def vbcast(vec, e):
        idx = jnp.full((16, 1), e, jnp.int32)
        dn = lax.GatherDimensionNumbers(offset_dims=(),
                                        collapsed_slice_dims=(0,),
                                        start_index_map=(0,))
        return lax.gather(vec, idx, dn, (1,),
                          mode=lax.GatherScatterMode.PROMISE_IN_BOUNDS)

    def zero_row(r, carry):
        for j in range(D // 16):
            g0[r, pl.ds(j * 16, 16)] = zero16
        return carry

    lax.fori_loop(0, C, zero_row, 0)

    off = 0
    for sz in WCH:
        pltpu.sync_copy(g0.at[pl.ds(0, sz)],
                        acc.at[pl.ds(s * NPT + off, sz)])
        off += sz
    plsc.subcore_barrier()

    def idx_fetch(i, q):
        pltpu.async_copy(aux_hbm.at[wid, i], aux[q], semi[q])

    def idx_wait(i, q):
        pltpu.make_async_copy(aux_hbm.at[wid, i], aux[q], semi[q]).wait()

    def gather_issue(b, q):
        pltpu.async_copy(x_hbm.at[aux[q].at[1]], gbuf[b], semg[b])

    def body(i, phase):
        b, q = phase % 2, phase
        b1, q1 = (phase + 1) % 2, (phase + 1) % 4
        pltpu.make_async_copy(x_hbm.at[aux[q].at[1]], gbuf[b], semg[b]).wait()

        @pl.when(i >= 2)
        def _():
            pltpu.make_async_copy(sbuf[b], acc.at[aux[q].at[0]], sems[b]).wait()

        @pl.when(i + 1 <= ncv - 1)
        def _():
            idx_wait(i + 1, q1)
            gather_issue(b1, q1)

        @pl.when((i >= 2) & (i + 2 <= ncv - 1))
        def _():
            idx_fetch(i + 2, (phase + 2) % 4)

        def scale_group(gidx, carry2):
            base = jnp.minimum(gidx * 16, C - 16)
            w16 = lax.bitcast_convert_type(aux[q][2, pl.ds(base, 16)],
                                           jnp.float32)
            for e in range(16):
                wb = vbcast(w16, e)
                r = base + e
                for j in range(D // 16):
                    sl = pl.ds(j * 16, 16)
                    sbuf[b][r, sl] = gbuf[b][r, sl] * wb
            return carry2

        lax.fori_loop(0, (C + 15) // 16, scale_group, 0)
        pltpu.async_copy(sbuf[b], acc.at[aux[q].at[0]], sems[b], add=True)

    for i in range(4):
        idx_fetch(i, i)
    idx_wait(0, 0)
    gather_issue(0, 0)

    def outer(kk, carry):
        for u in range(4):
            body(kk * 4 + u, u)
        return carry

    lax.fori_loop(0, ncv // 4, outer, 0)

    pltpu.make_async_copy(sbuf[0], acc.at[aux[0].at[0]], sems[0]).wait()
    pltpu.make_async_copy(sbuf[1], acc.at[aux[1].at[0]], sems[1]).wait()

    plsc.subcore_barrier()

    off = 0
    for sz in WCH:
        rb = s * NPT + off
        pltpu.sync_copy(acc.at[pl.ds(rb, sz)], g0.at[pl.ds(0, sz)])
        pltpu.sync_copy(g0.at[pl.ds(0, sz)], parts_hbm.at[c, pl.ds(rb, sz)])
        off += sz


def _sc_scatter(aux, x):
    mesh = plsc.VectorSubcoreMesh(core_axis_name="c", subcore_axis_name="s",
                                  num_cores=NC, num_subcores=NS)
    return pl.kernel(
        _sc_body,
        out_type=jax.ShapeDtypeStruct((NC, N_PAD, D), jnp.float32),
        mesh=mesh,
        scratch_types=[
            pltpu.VMEM_SHARED((N_PAD, D), jnp.float32),
            pltpu.VMEM((C, D), jnp.float32),
            pltpu.VMEM((C, D), jnp.float32),
            pltpu.VMEM((C, D), jnp.float32),
            pltpu.VMEM((C, D), jnp.float32),
            pltpu.VMEM((3, C), jnp.int32),
            pltpu.VMEM((3, C), jnp.int32),
            pltpu.VMEM((3, C), jnp.int32),
            pltpu.VMEM((3, C), jnp.int32),
            pltpu.SemaphoreType.DMA,
            pltpu.SemaphoreType.DMA,
            pltpu.SemaphoreType.DMA,
            pltpu.SemaphoreType.DMA,
            pltpu.SemaphoreType.DMA,
            pltpu.SemaphoreType.DMA,
            pltpu.SemaphoreType.DMA,
            pltpu.SemaphoreType.DMA,
        ],
    )(aux, x)


def _combine_body(p_ref, d_ref, o_ref):
    o_ref[...] = (p_ref[0] + p_ref[1]) * d_ref[...]


def _combine(parts, deg2d):
    bn = 2000
    return pl.pallas_call(
        _combine_body,
        out_shape=jax.ShapeDtypeStruct((N, D), jnp.float32),
        grid=(N // bn,),
        in_specs=[
            pl.BlockSpec((NC, bn, D), lambda i: (0, i, 0)),
            pl.BlockSpec((bn, 1), lambda i: (i, 0)),
        ],
        out_specs=pl.BlockSpec((bn, D), lambda i: (i, 0)),
    )(parts, deg2d)


def kernel(x, edge_index, edge_weight, deg_inv):
    row = edge_index[0].astype(jnp.int32)
    col = edge_index[1].astype(jnp.int32)
    wbits = lax.bitcast_convert_type(edge_weight.astype(jnp.float32), jnp.int32)
    pad = E_PAD - E
    zpad = jnp.zeros((pad,), jnp.int32)
    n0 = NS * CH0 * C

    def split(arr):
        arrp = jnp.concatenate([arr, zpad])
        return (arrp[:n0].reshape(NS, CH0, C), arrp[n0:].reshape(NS, CH1, C))

    r0, r1 = split(row)
    c0, c1 = split(col)
    w0, w1 = split(wbits)
    aux0 = jnp.stack([r0, c0, w0], axis=2)
    aux1 = jnp.stack([r1, c1, w1], axis=2)
    aux = jnp.zeros((NW, CHMAX, 3, C), jnp.int32)
    aux = aux.at[:NS, :CH0].set(aux0).at[NS:, :CH1].set(aux1)
    parts = _sc_scatter(aux, x)
    return _combine(parts, deg_inv[:, None])


# speedup vs baseline: 1.0700x; 1.0362x over previous
import jax
import jax.numpy as jnp
from jax import lax
from jax.experimental import pallas as pl
from jax.experimental.pallas import tpu as pltpu
from jax.experimental.pallas import tpu_sc as plsc

N = 10000
D = 128
E = 320000
NC, NS = 2, 16
NW = NC * NS
C = 88
CH0 = 80
CH1 = 148
CHMAX = max(CH0, CH1)
E_PAD = NS * (CH0 + CH1) * C
N_PAD = 10112
NPT = N_PAD // NS
WCH = [C] * 7 + [16]


def _sc_body(aux_hbm, x_hbm, parts_hbm,
             acc, g0, g1, s0, s1, a0, a1, a2, a3,
             semg0, semg1, sems0, sems1, semi0, semi1, semi2, semi3):
    c = lax.axis_index("c")
    s = lax.axis_index("s")
    wid = c * NS + s
    ncv = jnp.where(c == 0, CH0, CH1)
    gbuf = (g0, g1)
    sbuf = (s0, s1)
    aux = (a0, a1, a2, a3)
    semg = (semg0, semg1)
    sems = (sems0, sems1)
    semi = (semi0, semi1, semi2, semi3)

    zero16 = jnp.zeros((16,), jnp.float32)

    def vbcast(vec, e):
        idx = jnp.full((16, 1), e, jnp.int32)
        dn = lax.GatherDimensionNumbers(offset_dims=(),
                                        collapsed_slice_dims=(0,),
                                        start_index_map=(0,))
        return lax.gather(vec, idx, dn, (1,),
                          mode=lax.GatherScatterMode.PROMISE_IN_BOUNDS)

    def zero_row(r, carry):
        for j in range(D // 16):
            g0[r, pl.ds(j * 16, 16)] = zero16
        return carry

    lax.fori_loop(0, C, zero_row, 0)

    off = 0
    for sz in WCH:
        pltpu.sync_copy(g0.at[pl.ds(0, sz)],
                        acc.at[pl.ds(s * NPT + off, sz)])
        off += sz
    plsc.subcore_barrier()

    def idx_fetch(i, q):
        pltpu.async_copy(aux_hbm.at[wid, i], aux[q], semi[q])

    def idx_wait(i, q):
        pltpu.make_async_copy(aux_hbm.at[wid, i], aux[q], semi[q]).wait()

    def gather_issue(b, q):
        pltpu.async_copy(x_hbm.at[aux[q].at[1]], gbuf[b], semg[b])

    def body(i, phase):
        b, q = phase % 2, phase
        b1, q1 = (phase + 1) % 2, (phase + 1) % 4
        pltpu.make_async_copy(x_hbm.at[aux[q].at[1]], gbuf[b], semg[b]).wait()

        @pl.when(i >= 2)
        def _():
            pltpu.make_async_copy(sbuf[b], acc.at[aux[q].at[0]], sems[b]).wait()

        @pl.when(i + 1 <= ncv - 1)
        def _():
            idx_wait(i + 1, q1)
            gather_issue(b1, q1)

        @pl.when((i >= 2) & (i + 2 <= ncv - 1))
        def _():
            idx_fetch(i + 2, (phase + 2) % 4)

        def scale_group(gidx, carry2):
            base = jnp.minimum(gidx * 16, C - 16)
            w16 = lax.bitcast_convert_type(aux[q][2, pl.ds(base, 16)],
                                           jnp.float32)
            for e in range(16):
                wb = vbcast(w16, e)
                r = base + e
                for j in range(D // 16):
                    sl = pl.ds(j * 16, 16)
                    sbuf[b][r, sl] = gbuf[b][r, sl] * wb
            return carry2

        lax.fori_loop(0, (C + 15) // 16, scale_group, 0)
        pltpu.async_copy(sbuf[b], acc.at[aux[q].at[0]], sems[b], add=True)

    for i in range(4):
        idx_fetch(i, i)
    idx_wait(0, 0)
    gather_issue(0, 0)

    def outer(kk, carry):
        for u in range(4):
            body(kk * 4 + u, u)
        return carry

    lax.fori_loop(0, ncv // 4, outer, 0)

    pltpu.make_async_copy(sbuf[0], acc.at[aux[0].at[0]], sems[0]).wait()
    pltpu.make_async_copy(sbuf[1], acc.at[aux[1].at[0]], sems[1]).wait()

    plsc.subcore_barrier()

    off = 0
    for sz in WCH:
        rb = s * NPT + off
        pltpu.sync_copy(acc.at[pl.ds(rb, sz)], g0.at[pl.ds(0, sz)])
        pltpu.sync_copy(g0.at[pl.ds(0, sz)], parts_hbm.at[c, pl.ds(rb, sz)])
        off += sz


def _sc_scatter(aux, x):
    mesh = plsc.VectorSubcoreMesh(core_axis_name="c", subcore_axis_name="s",
                                  num_cores=NC, num_subcores=NS)
    return pl.kernel(
        _sc_body,
        out_type=jax.ShapeDtypeStruct((NC, N_PAD, D), jnp.float32),
        mesh=mesh,
        scratch_types=[
            pltpu.VMEM_SHARED((N_PAD, D), jnp.float32),
            pltpu.VMEM((C, D), jnp.float32),
            pltpu.VMEM((C, D), jnp.float32),
            pltpu.VMEM((C, D), jnp.float32),
            pltpu.VMEM((C, D), jnp.float32),
            pltpu.VMEM((3, C), jnp.int32),
            pltpu.VMEM((3, C), jnp.int32),
            pltpu.VMEM((3, C), jnp.int32),
            pltpu.VMEM((3, C), jnp.int32),
            pltpu.SemaphoreType.DMA,
            pltpu.SemaphoreType.DMA,
            pltpu.SemaphoreType.DMA,
            pltpu.SemaphoreType.DMA,
            pltpu.SemaphoreType.DMA,
            pltpu.SemaphoreType.DMA,
            pltpu.SemaphoreType.DMA,
            pltpu.SemaphoreType.DMA,
        ],
    )(aux, x)


def _combine_body(p_ref, d_ref, o_ref):
    o_ref[...] = (p_ref[0] + p_ref[1]) * d_ref[...]


def _combine(parts, deg2d):
    bn = 2000
    return pl.pallas_call(
        _combine_body,
        out_shape=jax.ShapeDtypeStruct((N, D), jnp.float32),
        grid=(N // bn,),
        in_specs=[
            pl.BlockSpec((NC, bn, D), lambda i: (0, i, 0)),
            pl.BlockSpec((bn, 1), lambda i: (i, 0)),
        ],
        out_specs=pl.BlockSpec((bn, D), lambda i: (i, 0)),
    )(parts, deg2d)


def kernel(x, edge_index, edge_weight, deg_inv):
    row = edge_index[0].astype(jnp.int32)
    col = edge_index[1].astype(jnp.int32)
    wbits = lax.bitcast_convert_type(edge_weight.astype(jnp.float32), jnp.int32)
    pad = E_PAD - E
    zpad = jnp.zeros((pad,), jnp.int32)
    n0 = NS * CH0 * C

    def split(arr):
        arrp = jnp.concatenate([arr, zpad])
        return (arrp[:n0].reshape(NS, CH0, C), arrp[n0:].reshape(NS, CH1, C))

    r0, r1 = split(row)
    c0, c1 = split(col)
    w0, w1 = split(wbits)
    aux0 = jnp.stack([r0, c0, w0], axis=2)
    aux1 = jnp.stack([r1, c1, w1], axis=2)
    aux = jnp.zeros((NW, CHMAX, 3, C), jnp.int32)
    aux = aux.at[:NS, :CH0].set(aux0).at[NS:, :CH1].set(aux1)
    parts = _sc_scatter(aux, x)
    return _combine(parts, deg_inv[:, None])


# speedup vs baseline: 1.0844x; 1.0135x over previous
import jax
import jax.numpy as jnp
from jax import lax
from jax.experimental import pallas as pl
from jax.experimental.pallas import tpu as pltpu
from jax.experimental.pallas import tpu_sc as plsc

N = 10000
D = 128
E = 320000
NC, NS = 2, 16
NW = NC * NS
C = 88
CH0 = 84
CH1 = 144
CHMAX = max(CH0, CH1)
E_PAD = NS * (CH0 + CH1) * C
N_PAD = 10112
NPT = N_PAD // NS
WCH = [C] * 7 + [16]


def _sc_body(aux_hbm, x_hbm, parts_hbm,
             acc, g0, g1, s0, s1, a0, a1, a2, a3,
             semg0, semg1, sems0, sems1, semi0, semi1, semi2, semi3):
    c = lax.axis_index("c")
    s = lax.axis_index("s")
    wid = c * NS + s
    ncv = jnp.where(c == 0, CH0, CH1)
    gbuf = (g0, g1)
    sbuf = (s0, s1)
    aux = (a0, a1, a2, a3)
    semg = (semg0, semg1)
    sems = (sems0, sems1)
    semi = (semi0, semi1, semi2, semi3)

    zero16 = jnp.zeros((16,), jnp.float32)

    def vbcast(vec, e):
        idx = jnp.full((16, 1), e, jnp.int32)
        dn = lax.GatherDimensionNumbers(offset_dims=(),
                                        collapsed_slice_dims=(0,),
                                        start_index_map=(0,))
        return lax.gather(vec, idx, dn, (1,),
                          mode=lax.GatherScatterMode.PROMISE_IN_BOUNDS)

    def zero_row(r, carry):
        for j in range(D // 16):
            g0[r, pl.ds(j * 16, 16)] = zero16
        return carry

    lax.fori_loop(0, C, zero_row, 0)

    off = 0
    for sz in WCH:
        pltpu.sync_copy(g0.at[pl.ds(0, sz)],
                        acc.at[pl.ds(s * NPT + off, sz)])
        off += sz
    plsc.subcore_barrier()

    def idx_fetch(i, q):
        pltpu.async_copy(aux_hbm.at[wid, i], aux[q], semi[q])

    def idx_wait(i, q):
        pltpu.make_async_copy(aux_hbm.at[wid, i], aux[q], semi[q]).wait()

    def gather_issue(b, q):
        pltpu.async_copy(x_hbm.at[aux[q].at[1]], gbuf[b], semg[b])

    def body(i, phase):
        b, q = phase % 2, phase
        b1, q1 = (phase + 1) % 2, (phase + 1) % 4
        pltpu.make_async_copy(x_hbm.at[aux[q].at[1]], gbuf[b], semg[b]).wait()

        @pl.when(i >= 2)
        def _():
            pltpu.make_async_copy(sbuf[b], acc.at[aux[q].at[0]], sems[b]).wait()

        @pl.when(i + 1 <= ncv - 1)
        def _():
            idx_wait(i + 1, q1)
            gather_issue(b1, q1)

        @pl.when((i >= 2) & (i + 2 <= ncv - 1))
        def _():
            idx_fetch(i + 2, (phase + 2) % 4)

        def scale_group(gidx, carry2):
            base = jnp.minimum(gidx * 16, C - 16)
            w16 = lax.bitcast_convert_type(aux[q][2, pl.ds(base, 16)],
                                           jnp.float32)
            for e in range(16):
                wb = vbcast(w16, e)
                r = base + e
                for j in range(D // 16):
                    sl = pl.ds(j * 16, 16)
                    sbuf[b][r, sl] = gbuf[b][r, sl] * wb
            return carry2

        lax.fori_loop(0, (C + 15) // 16, scale_group, 0)
        pltpu.async_copy(sbuf[b], acc.at[aux[q].at[0]], sems[b], add=True)

    for i in range(4):
        idx_fetch(i, i)
    idx_wait(0, 0)
    gather_issue(0, 0)

    def outer(kk, carry):
        for u in range(4):
            body(kk * 4 + u, u)
        return carry

    lax.fori_loop(0, ncv // 4, outer, 0)

    pltpu.make_async_copy(sbuf[0], acc.at[aux[0].at[0]], sems[0]).wait()
    pltpu.make_async_copy(sbuf[1], acc.at[aux[1].at[0]], sems[1]).wait()

    plsc.subcore_barrier()

    off = 0
    for sz in WCH:
        rb = s * NPT + off
        pltpu.sync_copy(acc.at[pl.ds(rb, sz)], g0.at[pl.ds(0, sz)])
        pltpu.sync_copy(g0.at[pl.ds(0, sz)], parts_hbm.at[c, pl.ds(rb, sz)])
        off += sz


def _sc_scatter(aux, x):
    mesh = plsc.VectorSubcoreMesh(core_axis_name="c", subcore_axis_name="s",
                                  num_cores=NC, num_subcores=NS)
    return pl.kernel(
        _sc_body,
        out_type=jax.ShapeDtypeStruct((NC, N_PAD, D), jnp.float32),
        mesh=mesh,
        scratch_types=[
            pltpu.VMEM_SHARED((N_PAD, D), jnp.float32),
            pltpu.VMEM((C, D), jnp.float32),
            pltpu.VMEM((C, D), jnp.float32),
            pltpu.VMEM((C, D), jnp.float32),
            pltpu.VMEM((C, D), jnp.float32),
            pltpu.VMEM((3, C), jnp.int32),
            pltpu.VMEM((3, C), jnp.int32),
            pltpu.VMEM((3, C), jnp.int32),
            pltpu.VMEM((3, C), jnp.int32),
            pltpu.SemaphoreType.DMA,
            pltpu.SemaphoreType.DMA,
            pltpu.SemaphoreType.DMA,
            pltpu.SemaphoreType.DMA,
            pltpu.SemaphoreType.DMA,
            pltpu.SemaphoreType.DMA,
            pltpu.SemaphoreType.DMA,
            pltpu.SemaphoreType.DMA,
        ],
    )(aux, x)


def _combine_body(p_ref, d_ref, o_ref):
    o_ref[...] = (p_ref[0] + p_ref[1]) * d_ref[...]


def _combine(parts, deg2d):
    bn = 2000
    return pl.pallas_call(
        _combine_body,
        out_shape=jax.ShapeDtypeStruct((N, D), jnp.float32),
        grid=(N // bn,),
        in_specs=[
            pl.BlockSpec((NC, bn, D), lambda i: (0, i, 0)),
            pl.BlockSpec((bn, 1), lambda i: (i, 0)),
        ],
        out_specs=pl.BlockSpec((bn, D), lambda i: (i, 0)),
    )(parts, deg2d)


def kernel(x, edge_index, edge_weight, deg_inv):
    row = edge_index[0].astype(jnp.int32)
    col = edge_index[1].astype(jnp.int32)
    wbits = lax.bitcast_convert_type(edge_weight.astype(jnp.float32), jnp.int32)
    pad = E_PAD - E
    zpad = jnp.zeros((pad,), jnp.int32)
    n0 = NS * CH0 * C

    def split(arr):
        arrp = jnp.concatenate([arr, zpad])
        return (arrp[:n0].reshape(NS, CH0, C), arrp[n0:].reshape(NS, CH1, C))

    r0, r1 = split(row)
    c0, c1 = split(col)
    w0, w1 = split(wbits)
    aux0 = jnp.stack([r0, c0, w0], axis=2)
    aux1 = jnp.stack([r1, c1, w1], axis=2)
    aux = jnp.zeros((NW, CHMAX, 3, C), jnp.int32)
    aux = aux.at[:NS, :CH0].set(aux0).at[NS:, :CH1].set(aux1)
    parts = _sc_scatter(aux, x)
    return _combine(parts, deg_inv[:, None])


# speedup vs baseline: 1.1218x; 1.0345x over previous
import jax
import jax.numpy as jnp
from jax import lax
from jax.experimental import pallas as pl
from jax.experimental.pallas import tpu as pltpu
from jax.experimental.pallas import tpu_sc as plsc

N = 10000
D = 128
E = 320000
NC, NS = 2, 16
NW = NC * NS
C = 88
CH0 = 92
CH1 = 136
CHMAX = max(CH0, CH1)
E_PAD = NS * (CH0 + CH1) * C
N_PAD = 10112
NPT = N_PAD // NS
WCH = [C] * 7 + [16]


def _sc_body(aux_hbm, x_hbm, parts_hbm,
             acc, g0, g1, s0, s1, a0, a1, a2, a3,
             semg0, semg1, sems0, sems1, semi0, semi1, semi2, semi3):
    c = lax.axis_index("c")
    s = lax.axis_index("s")
    wid = c * NS + s
    ncv = jnp.where(c == 0, CH0, CH1)
    gbuf = (g0, g1)
    sbuf = (s0, s1)
    aux = (a0, a1, a2, a3)
    semg = (semg0, semg1)
    sems = (sems0, sems1)
    semi = (semi0, semi1, semi2, semi3)

    zero16 = jnp.zeros((16,), jnp.float32)

    def vbcast(vec, e):
        idx = jnp.full((16, 1), e, jnp.int32)
        dn = lax.GatherDimensionNumbers(offset_dims=(),
                                        collapsed_slice_dims=(0,),
                                        start_index_map=(0,))
        return lax.gather(vec, idx, dn, (1,),
                          mode=lax.GatherScatterMode.PROMISE_IN_BOUNDS)

    def zero_row(r, carry):
        for j in range(D // 16):
            g0[r, pl.ds(j * 16, 16)] = zero16
        return carry

    lax.fori_loop(0, C, zero_row, 0)

    off = 0
    for sz in WCH:
        pltpu.sync_copy(g0.at[pl.ds(0, sz)],
                        acc.at[pl.ds(s * NPT + off, sz)])
        off += sz
    plsc.subcore_barrier()

    def idx_fetch(i, q):
        pltpu.async_copy(aux_hbm.at[wid, i], aux[q], semi[q])

    def idx_wait(i, q):
        pltpu.make_async_copy(aux_hbm.at[wid, i], aux[q], semi[q]).wait()

    def gather_issue(b, q):
        pltpu.async_copy(x_hbm.at[aux[q].at[1]], gbuf[b], semg[b])

    def body(i, phase):
        b, q = phase % 2, phase
        b1, q1 = (phase + 1) % 2, (phase + 1) % 4
        pltpu.make_async_copy(x_hbm.at[aux[q].at[1]], gbuf[b], semg[b]).wait()

        @pl.when(i >= 2)
        def _():
            pltpu.make_async_copy(sbuf[b], acc.at[aux[q].at[0]], sems[b]).wait()

        @pl.when(i + 1 <= ncv - 1)
        def _():
            idx_wait(i + 1, q1)
            gather_issue(b1, q1)

        @pl.when((i >= 2) & (i + 2 <= ncv - 1))
        def _():
            idx_fetch(i + 2, (phase + 2) % 4)

        def scale_group(gidx, carry2):
            base = jnp.minimum(gidx * 16, C - 16)
            w16 = lax.bitcast_convert_type(aux[q][2, pl.ds(base, 16)],
                                           jnp.float32)
            for e in range(16):
                wb = vbcast(w16, e)
                r = base + e
                for j in range(D // 16):
                    sl = pl.ds(j * 16, 16)
                    sbuf[b][r, sl] = gbuf[b][r, sl] * wb
            return carry2

        lax.fori_loop(0, (C + 15) // 16, scale_group, 0)
        pltpu.async_copy(sbuf[b], acc.at[aux[q].at[0]], sems[b], add=True)

    for i in range(4):
        idx_fetch(i, i)
    idx_wait(0, 0)
    gather_issue(0, 0)

    def outer(kk, carry):
        for u in range(4):
            body(kk * 4 + u, u)
        return carry

    lax.fori_loop(0, ncv // 4, outer, 0)

    pltpu.make_async_copy(sbuf[0], acc.at[aux[0].at[0]], sems[0]).wait()
    pltpu.make_async_copy(sbuf[1], acc.at[aux[1].at[0]], sems[1]).wait()

    plsc.subcore_barrier()

    off = 0
    for sz in WCH:
        rb = s * NPT + off
        pltpu.sync_copy(acc.at[pl.ds(rb, sz)], g0.at[pl.ds(0, sz)])
        pltpu.sync_copy(g0.at[pl.ds(0, sz)], parts_hbm.at[c, pl.ds(rb, sz)])
        off += sz


def _sc_scatter(aux, x):
    mesh = plsc.VectorSubcoreMesh(core_axis_name="c", subcore_axis_name="s",
                                  num_cores=NC, num_subcores=NS)
    return pl.kernel(
        _sc_body,
        out_type=jax.ShapeDtypeStruct((NC, N_PAD, D), jnp.float32),
        mesh=mesh,
        scratch_types=[
            pltpu.VMEM_SHARED((N_PAD, D), jnp.float32),
            pltpu.VMEM((C, D), jnp.float32),
            pltpu.VMEM((C, D), jnp.float32),
            pltpu.VMEM((C, D), jnp.float32),
            pltpu.VMEM((C, D), jnp.float32),
            pltpu.VMEM((3, C), jnp.int32),
            pltpu.VMEM((3, C), jnp.int32),
            pltpu.VMEM((3, C), jnp.int32),
            pltpu.VMEM((3, C), jnp.int32),
            pltpu.SemaphoreType.DMA,
            pltpu.SemaphoreType.DMA,
            pltpu.SemaphoreType.DMA,
            pltpu.SemaphoreType.DMA,
            pltpu.SemaphoreType.DMA,
            pltpu.SemaphoreType.DMA,
            pltpu.SemaphoreType.DMA,
            pltpu.SemaphoreType.DMA,
        ],
    )(aux, x)


def _combine_body(p_ref, d_ref, o_ref):
    o_ref[...] = (p_ref[0] + p_ref[1]) * d_ref[...]


def _combine(parts, deg2d):
    bn = 2000
    return pl.pallas_call(
        _combine_body,
        out_shape=jax.ShapeDtypeStruct((N, D), jnp.float32),
        grid=(N // bn,),
        in_specs=[
            pl.BlockSpec((NC, bn, D), lambda i: (0, i, 0)),
            pl.BlockSpec((bn, 1), lambda i: (i, 0)),
        ],
        out_specs=pl.BlockSpec((bn, D), lambda i: (i, 0)),
    )(parts, deg2d)


def kernel(x, edge_index, edge_weight, deg_inv):
    row = edge_index[0].astype(jnp.int32)
    col = edge_index[1].astype(jnp.int32)
    wbits = lax.bitcast_convert_type(edge_weight.astype(jnp.float32), jnp.int32)
    pad = E_PAD - E
    zpad = jnp.zeros((pad,), jnp.int32)
    n0 = NS * CH0 * C

    def split(arr):
        arrp = jnp.concatenate([arr, zpad])
        return (arrp[:n0].reshape(NS, CH0, C), arrp[n0:].reshape(NS, CH1, C))

    r0, r1 = split(row)
    c0, c1 = split(col)
    w0, w1 = split(wbits)
    aux0 = jnp.stack([r0, c0, w0], axis=2)
    aux1 = jnp.stack([r1, c1, w1], axis=2)
    aux = jnp.zeros((NW, CHMAX, 3, C), jnp.int32)
    aux = aux.at[:NS, :CH0].set(aux0).at[NS:, :CH1].set(aux1)
    parts = _sc_scatter(aux, x)
    return _combine(parts, deg_inv[:, None])
